# baseline (speedup 1.0000x reference)
import functools

import jax
import jax.numpy as jnp
from jax import lax
from jax.experimental import pallas as pl
from jax.experimental.pallas import tpu as pltpu
from jax.experimental.pallas import tpu_sc as plsc

V = 1000001
D = 64
B = 4096
L = 200
N = B * L
NC = 2
NS = 16
NW = NC * NS
PER_W = N // NW
CH = 128
NCHUNK = PER_W // CH

BLK2 = 8192
CE = BLK2 * 2 * D
G1 = (V * D + CE - 1) // CE


def _mm_body(x_ref, w1_ref, b1_ref, w2_ref, b2_ref, o_ref):
    z = jnp.zeros((D, D), jnp.float32)
    w1d = jnp.concatenate(
        [jnp.concatenate([w1_ref[...], z], axis=1),
         jnp.concatenate([z, w1_ref[...]], axis=1)], axis=0)
    w2d = jnp.concatenate(
        [jnp.concatenate([w2_ref[...], z], axis=1),
         jnp.concatenate([z, w2_ref[...]], axis=1)], axis=0)
    b1d = jnp.concatenate([b1_ref[...], b1_ref[...]], axis=1)
    b2d = jnp.concatenate([b2_ref[...], b2_ref[...]], axis=1)
    x = x_ref[...].reshape(BLK2, 2 * D)
    h = jnp.dot(x, w1d, preferred_element_type=jnp.float32) + b1d
    y = jnp.dot(h, w2d, preferred_element_type=jnp.float32) + b2d
    o_ref[...] = y.reshape(CE)


def _transform_table(tf, W1, b1, W2, b2):
    return pl.pallas_call(
        _mm_body,
        grid=(G1,),
        in_specs=[
            pl.BlockSpec((CE,), lambda i: (i,)),
            pl.BlockSpec((D, D), lambda i: (0, 0)),
            pl.BlockSpec((1, D), lambda i: (0, 0)),
            pl.BlockSpec((D, D), lambda i: (0, 0)),
            pl.BlockSpec((1, D), lambda i: (0, 0)),
        ],
        out_specs=pl.BlockSpec((CE,), lambda i: (i,)),
        out_shape=jax.ShapeDtypeStruct((V * D,), jnp.float32),
    )(tf, W1, b1.reshape(1, D), W2, b2.reshape(1, D))


def _sc_gather(table, ids):
    mesh = plsc.VectorSubcoreMesh(core_axis_name="c", subcore_axis_name="s")

    @functools.partial(
        pl.kernel,
        out_type=jax.ShapeDtypeStruct((N, D), jnp.float32),
        mesh=mesh,
        scratch_types=[
            pltpu.VMEM((PER_W,), jnp.int32),
            [pltpu.VMEM((CH, D), jnp.float32)] * 4,
            [pltpu.SemaphoreType.DMA] * 4,
            [pltpu.SemaphoreType.DMA] * 4,
        ],
        compiler_params=pltpu.CompilerParams(use_tc_tiling_on_sc=False),
    )
    def k(t_hbm, idx_hbm, out_hbm, idx_v, bufs, sgs, sws):
        wid = lax.axis_index("s") * NC + lax.axis_index("c")
        base = wid * PER_W
        pltpu.sync_copy(idx_hbm.at[pl.ds(base, PER_W)], idx_v)

        def body(j, carry):
            gathers = []
            for q in range(4):
                jq = 4 * j + q
                gathers.append(pltpu.async_copy(
                    t_hbm.at[idx_v.at[pl.ds(jq * CH, CH)]], bufs[q], sgs[q]))
            writes = []
            for q in range(4):
                jq = 4 * j + q
                gathers[q].wait()
                writes.append(pltpu.async_copy(
                    bufs[q], out_hbm.at[pl.ds(base + jq * CH, CH)], sws[q]))
            for q in range(4):
                writes[q].wait()
            return carry

        lax.fori_loop(0, NCHUNK // 4, body, 0)

    return k(table, ids)


def kernel(input_ids, emb_table, W1, b1, W2, b2):
    tf = emb_table.reshape(V * D)
    t = _transform_table(tf, W1, b1, W2, b2).reshape(V, D)
    ids = input_ids.reshape(N)
    g = _sc_gather(t, ids)
    return (g.reshape(B, L, D),)


# speedup vs baseline: 1.0350x; 1.0350x over previous
import functools

import jax
import jax.numpy as jnp
from jax import lax
from jax.experimental import pallas as pl
from jax.experimental.pallas import tpu as pltpu
from jax.experimental.pallas import tpu_sc as plsc

V = 1000001
D = 64
B = 4096
L = 200
N = B * L
NC = 2
NS = 16
NW = NC * NS
PER_W = N // NW
CH = 128
NCHUNK = PER_W // CH


def _sc_gather(table, ids):
    mesh = plsc.VectorSubcoreMesh(core_axis_name="c", subcore_axis_name="s")

    @functools.partial(
        pl.kernel,
        out_type=jax.ShapeDtypeStruct((N, D), jnp.float32),
        mesh=mesh,
        scratch_types=[
            pltpu.VMEM((PER_W,), jnp.int32),
            [pltpu.VMEM((CH, D), jnp.float32)] * 8,
            [pltpu.SemaphoreType.DMA] * 8,
            [pltpu.SemaphoreType.DMA] * 8,
        ],
        compiler_params=pltpu.CompilerParams(use_tc_tiling_on_sc=False),
    )
    def k(t_hbm, idx_hbm, out_hbm, idx_v, bufs, sgs, sws):
        wid = lax.axis_index("s") * NC + lax.axis_index("c")
        base = wid * PER_W
        pltpu.sync_copy(idx_hbm.at[pl.ds(base, PER_W)], idx_v)

        def body(j, carry):
            gathers = []
            for q in range(8):
                jq = 8 * j + q
                gathers.append(pltpu.async_copy(
                    t_hbm.at[idx_v.at[pl.ds(jq * CH, CH)]], bufs[q], sgs[q]))
            writes = []
            for q in range(8):
                jq = 8 * j + q
                gathers[q].wait()
                writes.append(pltpu.async_copy(
                    bufs[q], out_hbm.at[pl.ds(base + jq * CH, CH)], sws[q]))
            for q in range(8):
                writes[q].wait()
            return carry

        lax.fori_loop(0, NCHUNK // 8, body, 0)

    return k(table, ids)


BLK2 = 8192
CE = BLK2 * 2 * D
G2 = (N * D) // CE


def _mm_body(x_ref, w1_ref, b1_ref, w2_ref, b2_ref, o_ref):
    z = jnp.zeros((D, D), jnp.float32)
    w1d = jnp.concatenate(
        [jnp.concatenate([w1_ref[...], z], axis=1),
         jnp.concatenate([z, w1_ref[...]], axis=1)], axis=0)
    w2d = jnp.concatenate(
        [jnp.concatenate([w2_ref[...], z], axis=1),
         jnp.concatenate([z, w2_ref[...]], axis=1)], axis=0)
    b1d = jnp.concatenate([b1_ref[...], b1_ref[...]], axis=1)
    b2d = jnp.concatenate([b2_ref[...], b2_ref[...]], axis=1)
    x = x_ref[...].reshape(BLK2, 2 * D)
    h = jnp.dot(x, w1d, preferred_element_type=jnp.float32) + b1d
    y = jnp.dot(h, w2d, preferred_element_type=jnp.float32) + b2d
    o_ref[...] = y.reshape(CE)


def _final_mm(xf, W1, b1, W2, b2):
    return pl.pallas_call(
        _mm_body,
        grid=(G2,),
        in_specs=[
            pl.BlockSpec((CE,), lambda i: (i,)),
            pl.BlockSpec((D, D), lambda i: (0, 0)),
            pl.BlockSpec((1, D), lambda i: (0, 0)),
            pl.BlockSpec((D, D), lambda i: (0, 0)),
            pl.BlockSpec((1, D), lambda i: (0, 0)),
        ],
        out_specs=pl.BlockSpec((CE,), lambda i: (i,)),
        out_shape=jax.ShapeDtypeStruct((N * D,), jnp.float32),
    )(xf, W1, b1.reshape(1, D), W2, b2.reshape(1, D))


def kernel(input_ids, emb_table, W1, b1, W2, b2):
    ids = input_ids.reshape(N)
    g = _sc_gather(emb_table, ids)
    yf = _final_mm(g.reshape(N * D), W1, b1, W2, b2)
    return (yf.reshape(B, L, D),)
